# Initial kernel scaffold; baseline (speedup 1.0000x reference)
import jax
import jax.numpy as jnp
from jax.experimental import pallas as pl


def kernel(node_features, edge_index, edge_features, msg_W, msg_b, W_ih, W_hh, b_ih, b_hh, c1_W, c1_b, c2_W, c2_b):
    raise NotImplementedError("write your pallas kernel here")


# speedup vs baseline: 1.7055x; 1.7055x over previous
import functools
import jax
import jax.numpy as jnp
from jax import lax
from jax.experimental import pallas as pl
from jax.experimental.pallas import tpu as pltpu
from jax.experimental.pallas import tpu_sc as plsc

N = 10000
E = 320000
D = 128
DE = 16
NCLS = 8

NC = 2
NS = 16
NW = NC * NS
EPW = E // NW
CH = 80
NCHUNK = EPW // CH
NP = 10240
RPT = NP // NS
ZR = 128

BN = 2048
BE = 4000

_SQRT_HALF = 0.7071067811865476


def _gelu_sc(x):
    z = jnp.abs(x) * _SQRT_HALF
    t = 1.0 / (z * 0.3275911 + 1.0)
    poly = ((((1.061405429 * t - 1.453152027) * t + 1.421413741) * t
             - 0.284496736) * t + 0.254829592) * t
    erf = jnp.sign(x) * (1.0 - poly * jnp.exp(-(z * z)))
    return 0.5 * x * (1.0 + erf)


def _pre_nodes_body(nf_ref, w_ref, xs_ref, xt_ref):
    d = jnp.dot(nf_ref[...], w_ref[...], preferred_element_type=jnp.float32,
                precision=lax.Precision.HIGHEST)
    xs_ref[...] = d[:, :D]
    xt_ref[...] = d[:, D:]


def _pre_edges_body(ef_ref, w_ref, b_ref, p_ref):
    p_ref[...] = jnp.dot(ef_ref[...], w_ref[...],
                         preferred_element_type=jnp.float32,
                         precision=lax.Precision.HIGHEST) + b_ref[...]


def _gru_body(a0_ref, a1_ref, nf_ref, wih_ref, whh_ref, bih_ref, bhh_ref,
              cst_ref, ys_ref, yt_ref):
    agg = a0_ref[...] + a1_ref[...]
    h = nf_ref[...]
    gi = jnp.dot(agg, wih_ref[...], preferred_element_type=jnp.float32,
                 precision=lax.Precision.HIGHEST) + bih_ref[...]
    gh = jnp.dot(h, whh_ref[...], preferred_element_type=jnp.float32,
                 precision=lax.Precision.HIGHEST) + bhh_ref[...]
    r = jax.nn.sigmoid(gi[:, :D] + gh[:, :D])
    z = jax.nn.sigmoid(gi[:, D:2 * D] + gh[:, D:2 * D])
    n = jnp.tanh(gi[:, 2 * D:] + r * gh[:, 2 * D:])
    hn = (1.0 - z) * n + z * h
    y = jnp.dot(hn, cst_ref[...], preferred_element_type=jnp.float32,
                precision=lax.Precision.HIGHEST)
    ys_ref[...] = y[:, :DE]
    yt_ref[...] = y[:, DE:]


def _cls_body(s_ref, ef_ref, cet_ref, c1b_ref, c2wt_ref, c2b_ref, out_ref):
    x = s_ref[...] + jnp.dot(ef_ref[...], cet_ref[...],
                             preferred_element_type=jnp.float32,
                             precision=lax.Precision.HIGHEST) + c1b_ref[...]
    hgelu = _gelu_sc(x)
    out_ref[...] = jnp.dot(hgelu, c2wt_ref[...],
                           preferred_element_type=jnp.float32,
                           precision=lax.Precision.HIGHEST) + c2b_ref[...]


_MESH = plsc.VectorSubcoreMesh(core_axis_name="c", subcore_axis_name="s",
                               num_cores=NC, num_subcores=NS)


@functools.partial(
    pl.kernel,
    out_type=(jax.ShapeDtypeStruct((NP, D), jnp.float32),
              jax.ShapeDtypeStruct((NP, D), jnp.float32)),
    mesh=_MESH,
    scratch_types=[
        pltpu.VMEM((CH,), jnp.int32),
        pltpu.VMEM((CH,), jnp.int32),
        pltpu.VMEM((CH, D), jnp.float32),
        pltpu.VMEM((CH, D), jnp.float32),
        pltpu.VMEM((CH, D), jnp.float32),
        pltpu.VMEM((ZR, D), jnp.float32),
        pltpu.VMEM_SHARED((NP, D), jnp.float32),
        pltpu.SemaphoreType.DMA,
        pltpu.SemaphoreType.DMA,
        pltpu.SemaphoreType.DMA,
    ],
)
def _edge_a(xs_hbm, xt_hbm, p_hbm, src_hbm, tgt_hbm, out0, out1,
            idx_s, idx_t, bufS, bufT, bufP, zbuf, agg_sh, sem0, sem1, sem2):
    cid = lax.axis_index("c")
    sid = lax.axis_index("s")
    wid = sid * NC + cid

    def zrow(r, carry):
        for c8 in range(D // 16):
            zbuf[r, pl.ds(c8 * 16, 16)] = jnp.zeros((16,), jnp.float32)
        return carry

    lax.fori_loop(0, ZR, zrow, 0)
    for j in range(RPT // ZR):
        pltpu.sync_copy(zbuf, agg_sh.at[pl.ds(sid * RPT + j * ZR, ZR), :])
    plsc.subcore_barrier()

    base0 = wid * EPW

    def chunk(i, carry):
        base = base0 + i * CH
        pltpu.sync_copy(src_hbm.at[pl.ds(base, CH)], idx_s)
        pltpu.sync_copy(tgt_hbm.at[pl.ds(base, CH)], idx_t)
        cs = pltpu.async_copy(xs_hbm.at[idx_s], bufS, sem0)
        ct = pltpu.async_copy(xt_hbm.at[idx_t], bufT, sem1)
        cp = pltpu.async_copy(p_hbm.at[pl.ds(base, CH), :], bufP, sem2)
        cs.wait()
        ct.wait()
        cp.wait()

        def row(r, rc):
            for c8 in range(D // 16):
                sl = pl.ds(c8 * 16, 16)
                x = bufS[r, sl] + bufT[r, sl] + bufP[r, sl]
                bufS[r, sl] = _gelu_sc(x)
            return rc

        lax.fori_loop(0, CH, row, 0)
        pltpu.sync_copy(bufS, agg_sh.at[idx_t], add=True)
        return carry

    lax.fori_loop(0, NCHUNK, chunk, 0)
    plsc.subcore_barrier()

    for j in range(RPT // ZR):
        rows = pl.ds(sid * RPT + j * ZR, ZR)
        pltpu.sync_copy(agg_sh.at[rows, :], zbuf)

        @pl.when(cid == 0)
        def _():
            pltpu.sync_copy(zbuf, out0.at[rows, :])

        @pl.when(cid == 1)
        def _():
            pltpu.sync_copy(zbuf, out1.at[rows, :])


@functools.partial(
    pl.kernel,
    out_type=jax.ShapeDtypeStruct((E, DE), jnp.float32),
    mesh=_MESH,
    compiler_params=pltpu.CompilerParams(use_tc_tiling_on_sc=False),
    scratch_types=[
        pltpu.VMEM((CH,), jnp.int32),
        pltpu.VMEM((CH,), jnp.int32),
        pltpu.VMEM((CH, DE), jnp.float32),
        pltpu.VMEM((CH, DE), jnp.float32),
        pltpu.SemaphoreType.DMA,
        pltpu.SemaphoreType.DMA,
    ],
)
def _edge_b(ys_hbm, yt_hbm, src_hbm, tgt_hbm, s_out,
            idx_s, idx_t, bufA, bufB, sem0, sem1):
    cid = lax.axis_index("c")
    sid = lax.axis_index("s")
    wid = sid * NC + cid
    base0 = wid * EPW

    def chunk(i, carry):
        base = base0 + i * CH
        pltpu.sync_copy(src_hbm.at[pl.ds(base, CH)], idx_s)
        pltpu.sync_copy(tgt_hbm.at[pl.ds(base, CH)], idx_t)
        ca = pltpu.async_copy(ys_hbm.at[idx_s], bufA, sem0)
        cb = pltpu.async_copy(yt_hbm.at[idx_t], bufB, sem1)
        ca.wait()
        cb.wait()

        def row(r, rc):
            sl = pl.ds(0, DE)
            bufA[r, sl] = bufA[r, sl] + bufB[r, sl]
            return rc

        lax.fori_loop(0, CH, row, 0)
        pltpu.sync_copy(bufA, s_out.at[pl.ds(base, CH), :])
        return carry

    lax.fori_loop(0, NCHUNK, chunk, 0)


def kernel(node_features, edge_index, edge_features, msg_W, msg_b,
           W_ih, W_hh, b_ih, b_hh, c1_W, c1_b, c2_W, c2_b):
    src = edge_index[0]
    tgt = edge_index[1]
    nf_pad = jnp.pad(node_features, ((0, NP - N), (0, 0)))

    wst = jnp.concatenate([msg_W[:, :D].T, msg_W[:, D:2 * D].T], axis=1)
    wet = msg_W[:, 2 * D:].T
    wih_t = W_ih.T
    whh_t = W_hh.T
    cst = jnp.concatenate([c1_W[:, :D].T, c1_W[:, D:2 * D].T], axis=1)
    cet = c1_W[:, 2 * D:].T
    c2wt = c2_W.T

    xs, xt = pl.pallas_call(
        _pre_nodes_body,
        grid=(NP // BN,),
        in_specs=[pl.BlockSpec((BN, D), lambda i: (i, 0)),
                  pl.BlockSpec((D, 2 * D), lambda i: (0, 0))],
        out_specs=[pl.BlockSpec((BN, D), lambda i: (i, 0)),
                   pl.BlockSpec((BN, D), lambda i: (i, 0))],
        out_shape=[jax.ShapeDtypeStruct((NP, D), jnp.float32),
                   jax.ShapeDtypeStruct((NP, D), jnp.float32)],
    )(nf_pad, wst)

    p = pl.pallas_call(
        _pre_edges_body,
        grid=(E // BE,),
        in_specs=[pl.BlockSpec((BE, DE), lambda i: (i, 0)),
                  pl.BlockSpec((DE, D), lambda i: (0, 0)),
                  pl.BlockSpec((1, D), lambda i: (0, 0))],
        out_specs=pl.BlockSpec((BE, D), lambda i: (i, 0)),
        out_shape=jax.ShapeDtypeStruct((E, D), jnp.float32),
    )(edge_features, wet, msg_b.reshape(1, D))

    agg0, agg1 = _edge_a(xs, xt, p, src, tgt)

    ys, yt = pl.pallas_call(
        _gru_body,
        grid=(NP // BN,),
        in_specs=[pl.BlockSpec((BN, D), lambda i: (i, 0)),
                  pl.BlockSpec((BN, D), lambda i: (i, 0)),
                  pl.BlockSpec((BN, D), lambda i: (i, 0)),
                  pl.BlockSpec((D, 3 * D), lambda i: (0, 0)),
                  pl.BlockSpec((D, 3 * D), lambda i: (0, 0)),
                  pl.BlockSpec((1, 3 * D), lambda i: (0, 0)),
                  pl.BlockSpec((1, 3 * D), lambda i: (0, 0)),
                  pl.BlockSpec((D, 2 * DE), lambda i: (0, 0))],
        out_specs=[pl.BlockSpec((BN, DE), lambda i: (i, 0)),
                   pl.BlockSpec((BN, DE), lambda i: (i, 0))],
        out_shape=[jax.ShapeDtypeStruct((NP, DE), jnp.float32),
                   jax.ShapeDtypeStruct((NP, DE), jnp.float32)],
    )(agg0, agg1, nf_pad, wih_t, whh_t,
      b_ih.reshape(1, 3 * D), b_hh.reshape(1, 3 * D), cst)

    s = _edge_b(ys, yt, src, tgt)

    logits = pl.pallas_call(
        _cls_body,
        grid=(E // BE,),
        in_specs=[pl.BlockSpec((BE, DE), lambda i: (i, 0)),
                  pl.BlockSpec((BE, DE), lambda i: (i, 0)),
                  pl.BlockSpec((DE, DE), lambda i: (0, 0)),
                  pl.BlockSpec((1, DE), lambda i: (0, 0)),
                  pl.BlockSpec((DE, NCLS), lambda i: (0, 0)),
                  pl.BlockSpec((1, NCLS), lambda i: (0, 0))],
        out_specs=pl.BlockSpec((BE, NCLS), lambda i: (i, 0)),
        out_shape=jax.ShapeDtypeStruct((E, NCLS), jnp.float32),
    )(s, edge_features, cet, c1_b.reshape(1, DE), c2wt, c2_b.reshape(1, NCLS))

    return logits


# speedup vs baseline: 2.1217x; 1.2440x over previous
import functools
import jax
import jax.numpy as jnp
from jax import lax
from jax.experimental import pallas as pl
from jax.experimental.pallas import tpu as pltpu
from jax.experimental.pallas import tpu_sc as plsc

N = 10000
E = 320000
D = 128
DE = 16
NCLS = 8

NC = 2
NS = 16
NW = NC * NS
EPW = E // NW
CH = 80
NCHUNK = EPW // CH
NP = 10240
RPT = NP // NS
ZR = 128

BN = 2048
BE = 4000

_SQRT_HALF = 0.7071067811865476


def _gelu_sc(x):
    z = jnp.abs(x) * _SQRT_HALF
    t = 1.0 / (z * 0.3275911 + 1.0)
    poly = ((((1.061405429 * t - 1.453152027) * t + 1.421413741) * t
             - 0.284496736) * t + 0.254829592) * t
    erf = jnp.sign(x) * (1.0 - poly * jnp.exp(-(z * z)))
    return 0.5 * x * (1.0 + erf)


def _pre_nodes_body(nf_ref, w_ref, xs_ref, xt_ref):
    d = jnp.dot(nf_ref[...], w_ref[...], preferred_element_type=jnp.float32,
                precision=lax.Precision.HIGHEST)
    xs_ref[...] = d[:, :D]
    xt_ref[...] = d[:, D:]


def _pre_edges_body(ef_ref, w_ref, b_ref, p_ref):
    p_ref[...] = jnp.dot(ef_ref[...], w_ref[...],
                         preferred_element_type=jnp.float32,
                         precision=lax.Precision.HIGHEST) + b_ref[...]


def _gru_body(a0_ref, a1_ref, nf_ref, wih_ref, whh_ref, bih_ref, bhh_ref,
              cst_ref, ys_ref, yt_ref):
    agg = a0_ref[...] + a1_ref[...]
    h = nf_ref[...]
    gi = jnp.dot(agg, wih_ref[...], preferred_element_type=jnp.float32,
                 precision=lax.Precision.HIGHEST) + bih_ref[...]
    gh = jnp.dot(h, whh_ref[...], preferred_element_type=jnp.float32,
                 precision=lax.Precision.HIGHEST) + bhh_ref[...]
    r = jax.nn.sigmoid(gi[:, :D] + gh[:, :D])
    z = jax.nn.sigmoid(gi[:, D:2 * D] + gh[:, D:2 * D])
    n = jnp.tanh(gi[:, 2 * D:] + r * gh[:, 2 * D:])
    hn = (1.0 - z) * n + z * h
    y = jnp.dot(hn, cst_ref[...], preferred_element_type=jnp.float32,
                precision=lax.Precision.HIGHEST)
    ys_ref[...] = y[:, :DE]
    yt_ref[...] = y[:, DE:]


def _cls_body(s_ref, ef_ref, cet_ref, c1b_ref, c2wt_ref, c2b_ref, out_ref):
    x = s_ref[...] + jnp.dot(ef_ref[...], cet_ref[...],
                             preferred_element_type=jnp.float32,
                             precision=lax.Precision.HIGHEST) + c1b_ref[...]
    hgelu = _gelu_sc(x)
    out_ref[...] = jnp.dot(hgelu, c2wt_ref[...],
                           preferred_element_type=jnp.float32,
                           precision=lax.Precision.HIGHEST) + c2b_ref[...]


_MESH = plsc.VectorSubcoreMesh(core_axis_name="c", subcore_axis_name="s",
                               num_cores=NC, num_subcores=NS)


CHA = 40
NCHA = EPW // CHA


@functools.partial(
    pl.kernel,
    out_type=(jax.ShapeDtypeStruct((NP, D), jnp.float32),
              jax.ShapeDtypeStruct((NP, D), jnp.float32)),
    mesh=_MESH,
    scratch_types=[
        pltpu.VMEM((4, CHA), jnp.int32),
        pltpu.VMEM((4, CHA), jnp.int32),
        pltpu.VMEM((CHA, D), jnp.float32),
        pltpu.VMEM((CHA, D), jnp.float32),
        pltpu.VMEM((CHA, D), jnp.float32),
        pltpu.VMEM((CHA, D), jnp.float32),
        pltpu.VMEM((CHA, D), jnp.float32),
        pltpu.VMEM((CHA, D), jnp.float32),
        pltpu.VMEM_SHARED((NP, D), jnp.float32),
        pltpu.SemaphoreType.DMA,
        pltpu.SemaphoreType.DMA,
        pltpu.SemaphoreType.DMA,
        pltpu.SemaphoreType.DMA,
        pltpu.SemaphoreType.DMA,
        pltpu.SemaphoreType.DMA,
        pltpu.SemaphoreType.DMA,
        pltpu.SemaphoreType.DMA,
        pltpu.SemaphoreType.DMA,
        pltpu.SemaphoreType.DMA,
        pltpu.SemaphoreType.DMA,
        pltpu.SemaphoreType.DMA,
        pltpu.SemaphoreType.DMA,
        pltpu.SemaphoreType.DMA,
    ],
)
def _edge_a(xs_hbm, xt_hbm, p_hbm, src_hbm, tgt_hbm, out0, out1,
            ixs, ixt,
            bufSA, bufTA, bufPA, bufSB, bufTB, bufPB, agg_sh,
            sI0s, sI0t, sI1s, sI1t, sI2s, sI2t, sI3s, sI3t,
            sSA, sTA, sPA, sSB, sTB, sPB):
    cid = lax.axis_index("c")
    sid = lax.axis_index("s")
    wid = sid * NC + cid
    base0 = wid * EPW

    isems = ((sI0s, sI0t), (sI1s, sI1t), (sI2s, sI2t), (sI3s, sI3t))
    gsemA = (sSA, sTA, sPA)
    gsemB = (sSB, sTB, sPB)
    bufA = (bufSA, bufTA, bufPA)
    bufB = (bufSB, bufTB, bufPB)

    def zrow(r, carry):
        for c8 in range(D // 16):
            bufSA[r, pl.ds(c8 * 16, 16)] = jnp.zeros((16,), jnp.float32)
        return carry

    lax.fori_loop(0, CHA, zrow, 0)
    for j in range(RPT // CHA):
        pltpu.sync_copy(bufSA, agg_sh.at[pl.ds(sid * RPT + j * CHA, CHA), :])
    plsc.subcore_barrier()

    def start_idx(i, st):
        base = base0 + i * CHA
        pltpu.async_copy(src_hbm.at[pl.ds(base, CHA)], ixs.at[st], isems[st][0])
        pltpu.async_copy(tgt_hbm.at[pl.ds(base, CHA)], ixt.at[st], isems[st][1])

    def mid(i, st, bufs, gsems):
        dummy = src_hbm.at[pl.ds(base0, CHA)]
        pltpu.make_async_copy(dummy, ixs.at[st], isems[st][0]).wait()
        pltpu.make_async_copy(dummy, ixt.at[st], isems[st][1]).wait()
        base = base0 + i * CHA
        bS, bT, bP = bufs
        pltpu.async_copy(xs_hbm.at[ixs.at[st]], bS, gsems[0])
        pltpu.async_copy(xt_hbm.at[ixt.at[st]], bT, gsems[1])
        pltpu.async_copy(p_hbm.at[pl.ds(base, CHA), :], bP, gsems[2])

    def process(st, bufs, gsems):
        bS, bT, bP = bufs
        dummy = p_hbm.at[pl.ds(base0, CHA), :]
        pltpu.make_async_copy(dummy, bS, gsems[0]).wait()
        pltpu.make_async_copy(dummy, bT, gsems[1]).wait()
        pltpu.make_async_copy(dummy, bP, gsems[2]).wait()

        def row(r, rc):
            for c8 in range(D // 16):
                sl = pl.ds(c8 * 16, 16)
                x = bS[r, sl] + bT[r, sl] + bP[r, sl]
                bS[r, sl] = _gelu_sc(x)
            return rc

        lax.fori_loop(0, CHA, row, 0)
        pltpu.sync_copy(bS, agg_sh.at[ixt.at[st]], add=True)

    start_idx(0, 0)
    start_idx(1, 1)
    mid(0, 0, bufA, gsemA)
    start_idx(2, 2)
    mid(1, 1, bufB, gsemB)
    start_idx(3, 3)

    def body(k, carry):
        j = 4 * k
        process(0, bufA, gsemA)
        mid(j + 2, 2, bufA, gsemA)
        start_idx(j + 4, 0)
        process(1, bufB, gsemB)
        mid(j + 3, 3, bufB, gsemB)
        start_idx(j + 5, 1)
        process(2, bufA, gsemA)
        mid(j + 4, 0, bufA, gsemA)
        start_idx(j + 6, 2)
        process(3, bufB, gsemB)
        mid(j + 5, 1, bufB, gsemB)
        start_idx(j + 7, 3)
        return carry

    lax.fori_loop(0, NCHA // 4 - 1, body, 0)
    jt = NCHA - 6
    process(0, bufA, gsemA)
    mid(jt + 2, 2, bufA, gsemA)
    start_idx(jt + 4, 0)
    process(1, bufB, gsemB)
    mid(jt + 3, 3, bufB, gsemB)
    start_idx(jt + 5, 1)
    process(2, bufA, gsemA)
    mid(jt + 4, 0, bufA, gsemA)
    process(3, bufB, gsemB)
    mid(jt + 5, 1, bufB, gsemB)
    process(0, bufA, gsemA)
    process(1, bufB, gsemB)
    plsc.subcore_barrier()

    for j in range(RPT // CHA):
        rows = pl.ds(sid * RPT + j * CHA, CHA)
        pltpu.sync_copy(agg_sh.at[rows, :], bufSA)

        @pl.when(cid == 0)
        def _():
            pltpu.sync_copy(bufSA, out0.at[rows, :])

        @pl.when(cid == 1)
        def _():
            pltpu.sync_copy(bufSA, out1.at[rows, :])


@functools.partial(
    pl.kernel,
    out_type=jax.ShapeDtypeStruct((E, DE), jnp.float32),
    mesh=_MESH,
    compiler_params=pltpu.CompilerParams(use_tc_tiling_on_sc=False),
    scratch_types=[
        pltpu.VMEM((EPW,), jnp.int32),
        pltpu.VMEM((EPW,), jnp.int32),
        pltpu.VMEM((CH, DE), jnp.float32),
        pltpu.VMEM((CH, DE), jnp.float32),
        pltpu.VMEM((CH, DE), jnp.float32),
        pltpu.VMEM((CH, DE), jnp.float32),
        pltpu.SemaphoreType.DMA,
        pltpu.SemaphoreType.DMA,
        pltpu.SemaphoreType.DMA,
        pltpu.SemaphoreType.DMA,
        pltpu.SemaphoreType.DMA,
    ],
)
def _edge_b(ys_hbm, yt_hbm, src_hbm, tgt_hbm, s_out,
            ixs_all, ixt_all, bufAA, bufBA, bufAB, bufBB,
            sAA, sBA, sAB, sBB, sW):
    cid = lax.axis_index("c")
    sid = lax.axis_index("s")
    wid = sid * NC + cid
    base0 = wid * EPW

    pltpu.sync_copy(src_hbm.at[pl.ds(base0, EPW)], ixs_all)
    pltpu.sync_copy(tgt_hbm.at[pl.ds(base0, EPW)], ixt_all)

    def start(i, bA, bB, sA, sB):
        sl = pl.ds(i * CH, CH)
        dA = pltpu.async_copy(ys_hbm.at[ixs_all.at[sl]], bA, sA)
        dB = pltpu.async_copy(yt_hbm.at[ixt_all.at[sl]], bB, sB)
        return (dA, dB)

    def process(i, descs, bA, bB):
        for dsc in descs:
            dsc.wait()

        def row(r, rc):
            sl = pl.ds(0, DE)
            bA[r, sl] = bA[r, sl] + bB[r, sl]
            return rc

        lax.fori_loop(0, CH, row, 0)
        pltpu.async_copy(bA, s_out.at[pl.ds(base0 + i * CH, CH), :], sW).wait()

    slotA = (bufAA, bufBA)
    slotB = (bufAB, bufBB)
    semsA = (sAA, sBA)
    semsB = (sAB, sBB)
    dA = start(0, *slotA, *semsA)

    def pair(k, carry):
        j = 2 * k
        dB = start(j + 1, *slotB, *semsB)
        process(j, dA, *slotA)
        start(j + 2, *slotA, *semsA)
        process(j + 1, dB, *slotB)
        return carry

    lax.fori_loop(0, (NCHUNK - 1) // 2, pair, 0)
    process(NCHUNK - 1, dA, *slotA)


def kernel(node_features, edge_index, edge_features, msg_W, msg_b,
           W_ih, W_hh, b_ih, b_hh, c1_W, c1_b, c2_W, c2_b):
    src = edge_index[0]
    tgt = edge_index[1]
    nf_pad = jnp.pad(node_features, ((0, NP - N), (0, 0)))

    wst = jnp.concatenate([msg_W[:, :D].T, msg_W[:, D:2 * D].T], axis=1)
    wet = msg_W[:, 2 * D:].T
    wih_t = W_ih.T
    whh_t = W_hh.T
    cst = jnp.concatenate([c1_W[:, :D].T, c1_W[:, D:2 * D].T], axis=1)
    cet = c1_W[:, 2 * D:].T
    c2wt = c2_W.T

    xs, xt = pl.pallas_call(
        _pre_nodes_body,
        grid=(NP // BN,),
        in_specs=[pl.BlockSpec((BN, D), lambda i: (i, 0)),
                  pl.BlockSpec((D, 2 * D), lambda i: (0, 0))],
        out_specs=[pl.BlockSpec((BN, D), lambda i: (i, 0)),
                   pl.BlockSpec((BN, D), lambda i: (i, 0))],
        out_shape=[jax.ShapeDtypeStruct((NP, D), jnp.float32),
                   jax.ShapeDtypeStruct((NP, D), jnp.float32)],
    )(nf_pad, wst)

    p = pl.pallas_call(
        _pre_edges_body,
        grid=(E // BE,),
        in_specs=[pl.BlockSpec((BE, DE), lambda i: (i, 0)),
                  pl.BlockSpec((DE, D), lambda i: (0, 0)),
                  pl.BlockSpec((1, D), lambda i: (0, 0))],
        out_specs=pl.BlockSpec((BE, D), lambda i: (i, 0)),
        out_shape=jax.ShapeDtypeStruct((E, D), jnp.float32),
    )(edge_features, wet, msg_b.reshape(1, D))

    agg0, agg1 = _edge_a(xs, xt, p, src, tgt)

    ys, yt = pl.pallas_call(
        _gru_body,
        grid=(NP // BN,),
        in_specs=[pl.BlockSpec((BN, D), lambda i: (i, 0)),
                  pl.BlockSpec((BN, D), lambda i: (i, 0)),
                  pl.BlockSpec((BN, D), lambda i: (i, 0)),
                  pl.BlockSpec((D, 3 * D), lambda i: (0, 0)),
                  pl.BlockSpec((D, 3 * D), lambda i: (0, 0)),
                  pl.BlockSpec((1, 3 * D), lambda i: (0, 0)),
                  pl.BlockSpec((1, 3 * D), lambda i: (0, 0)),
                  pl.BlockSpec((D, 2 * DE), lambda i: (0, 0))],
        out_specs=[pl.BlockSpec((BN, DE), lambda i: (i, 0)),
                   pl.BlockSpec((BN, DE), lambda i: (i, 0))],
        out_shape=[jax.ShapeDtypeStruct((NP, DE), jnp.float32),
                   jax.ShapeDtypeStruct((NP, DE), jnp.float32)],
    )(agg0, agg1, nf_pad, wih_t, whh_t,
      b_ih.reshape(1, 3 * D), b_hh.reshape(1, 3 * D), cst)

    s = _edge_b(ys, yt, src, tgt)

    logits = pl.pallas_call(
        _cls_body,
        grid=(E // BE,),
        in_specs=[pl.BlockSpec((BE, DE), lambda i: (i, 0)),
                  pl.BlockSpec((BE, DE), lambda i: (i, 0)),
                  pl.BlockSpec((DE, DE), lambda i: (0, 0)),
                  pl.BlockSpec((1, DE), lambda i: (0, 0)),
                  pl.BlockSpec((DE, NCLS), lambda i: (0, 0)),
                  pl.BlockSpec((1, NCLS), lambda i: (0, 0))],
        out_specs=pl.BlockSpec((BE, NCLS), lambda i: (i, 0)),
        out_shape=jax.ShapeDtypeStruct((E, NCLS), jnp.float32),
    )(s, edge_features, cet, c1_b.reshape(1, DE), c2wt, c2_b.reshape(1, NCLS))

    return logits


# speedup vs baseline: 2.6534x; 1.2506x over previous
import functools
import jax
import jax.numpy as jnp
from jax import lax
from jax.experimental import pallas as pl
from jax.experimental.pallas import tpu as pltpu
from jax.experimental.pallas import tpu_sc as plsc

N = 10000
E = 320000
D = 128
DE = 16
NCLS = 8

NC = 2
NS = 16
NW = NC * NS
EPW = E // NW
CH = 80
NCHUNK = EPW // CH
NP = 10240
RPT = NP // NS
ZR = 128

BN = 2048
BE = 4000

_SQRT_HALF = 0.7071067811865476


def _gelu_sc(x):
    z = jnp.abs(x) * _SQRT_HALF
    t = 1.0 / (z * 0.3275911 + 1.0)
    poly = ((((1.061405429 * t - 1.453152027) * t + 1.421413741) * t
             - 0.284496736) * t + 0.254829592) * t
    erf = jnp.sign(x) * (1.0 - poly * jnp.exp(-(z * z)))
    return 0.5 * x * (1.0 + erf)


def _pre_nodes_body(nf_ref, w_ref, xs_ref, xt_ref):
    d = jnp.dot(nf_ref[...], w_ref[...], preferred_element_type=jnp.float32,
                precision=lax.Precision.HIGHEST)
    xs_ref[...] = d[:, :D]
    xt_ref[...] = d[:, D:]


def _pre_edges_body(efp_ref, w_ref, b_ref, p_ref):
    p_ref[...] = jnp.dot(efp_ref[...], w_ref[...],
                         preferred_element_type=jnp.float32,
                         precision=lax.Precision.HIGHEST) + b_ref[...]


def _gru_body(a0_ref, a1_ref, nf_ref, wih_ref, whh_ref, bih_ref, bhh_ref,
              cst_ref, ys_ref, yt_ref):
    agg = a0_ref[...] + a1_ref[...]
    h = nf_ref[...]
    gi = jnp.dot(agg, wih_ref[...], preferred_element_type=jnp.float32,
                 precision=lax.Precision.HIGHEST) + bih_ref[...]
    gh = jnp.dot(h, whh_ref[...], preferred_element_type=jnp.float32,
                 precision=lax.Precision.HIGHEST) + bhh_ref[...]
    r = jax.nn.sigmoid(gi[:, :D] + gh[:, :D])
    z = jax.nn.sigmoid(gi[:, D:2 * D] + gh[:, D:2 * D])
    n = jnp.tanh(gi[:, 2 * D:] + r * gh[:, 2 * D:])
    hn = (1.0 - z) * n + z * h
    y = jnp.dot(hn, cst_ref[...], preferred_element_type=jnp.float32,
                precision=lax.Precision.HIGHEST)
    ys_ref[...] = y[:, :DE]
    yt_ref[...] = y[:, DE:]


def _cls_body(s_ref, efp_ref, bd1_ref, c1b_ref, bd2_ref, c2b_ref, out_ref):
    x = s_ref[...] + jnp.dot(efp_ref[...], bd1_ref[...],
                             preferred_element_type=jnp.float32,
                             precision=lax.Precision.HIGHEST) + c1b_ref[...]
    hgelu = _gelu_sc(x)
    out_ref[...] = jnp.dot(hgelu, bd2_ref[...],
                           preferred_element_type=jnp.float32,
                           precision=lax.Precision.HIGHEST) + c2b_ref[...]


_MESH = plsc.VectorSubcoreMesh(core_axis_name="c", subcore_axis_name="s",
                               num_cores=NC, num_subcores=NS)


CHA = 40
NCHA = EPW // CHA


@functools.partial(
    pl.kernel,
    out_type=(jax.ShapeDtypeStruct((NP, D), jnp.float32),
              jax.ShapeDtypeStruct((NP, D), jnp.float32)),
    mesh=_MESH,
    scratch_types=[
        pltpu.VMEM((4, CHA), jnp.int32),
        pltpu.VMEM((4, CHA), jnp.int32),
        pltpu.VMEM((CHA, D), jnp.float32),
        pltpu.VMEM((CHA, D), jnp.float32),
        pltpu.VMEM((CHA, D), jnp.float32),
        pltpu.VMEM((CHA, D), jnp.float32),
        pltpu.VMEM((CHA, D), jnp.float32),
        pltpu.VMEM((CHA, D), jnp.float32),
        pltpu.VMEM_SHARED((NP, D), jnp.float32),
        pltpu.SemaphoreType.DMA,
        pltpu.SemaphoreType.DMA,
        pltpu.SemaphoreType.DMA,
        pltpu.SemaphoreType.DMA,
        pltpu.SemaphoreType.DMA,
        pltpu.SemaphoreType.DMA,
        pltpu.SemaphoreType.DMA,
        pltpu.SemaphoreType.DMA,
        pltpu.SemaphoreType.DMA,
        pltpu.SemaphoreType.DMA,
        pltpu.SemaphoreType.DMA,
        pltpu.SemaphoreType.DMA,
        pltpu.SemaphoreType.DMA,
        pltpu.SemaphoreType.DMA,
    ],
)
def _edge_a(xs_hbm, xt_hbm, p_hbm, src_hbm, tgt_hbm, out0, out1,
            ixs, ixt,
            bufSA, bufTA, bufPA, bufSB, bufTB, bufPB, agg_sh,
            sI0s, sI0t, sI1s, sI1t, sI2s, sI2t, sI3s, sI3t,
            sSA, sTA, sPA, sSB, sTB, sPB):
    cid = lax.axis_index("c")
    sid = lax.axis_index("s")
    wid = sid * NC + cid
    base0 = wid * EPW

    isems = ((sI0s, sI0t), (sI1s, sI1t), (sI2s, sI2t), (sI3s, sI3t))
    gsemA = (sSA, sTA, sPA)
    gsemB = (sSB, sTB, sPB)
    bufA = (bufSA, bufTA, bufPA)
    bufB = (bufSB, bufTB, bufPB)

    def zrow(r, carry):
        for c8 in range(D // 16):
            bufSA[r, pl.ds(c8 * 16, 16)] = jnp.zeros((16,), jnp.float32)
        return carry

    lax.fori_loop(0, CHA, zrow, 0)
    for j in range(RPT // CHA):
        pltpu.sync_copy(bufSA, agg_sh.at[pl.ds(sid * RPT + j * CHA, CHA), :])
    plsc.subcore_barrier()

    def start_idx(i, st):
        base = base0 + i * CHA
        pltpu.async_copy(src_hbm.at[pl.ds(base, CHA)], ixs.at[st], isems[st][0])
        pltpu.async_copy(tgt_hbm.at[pl.ds(base, CHA)], ixt.at[st], isems[st][1])

    def mid(i, st, bufs, gsems):
        dummy = src_hbm.at[pl.ds(base0, CHA)]
        pltpu.make_async_copy(dummy, ixs.at[st], isems[st][0]).wait()
        pltpu.make_async_copy(dummy, ixt.at[st], isems[st][1]).wait()
        base = base0 + i * CHA
        bS, bT, bP = bufs
        pltpu.async_copy(xs_hbm.at[ixs.at[st]], bS, gsems[0])
        pltpu.async_copy(xt_hbm.at[ixt.at[st]], bT, gsems[1])
        pltpu.async_copy(p_hbm.at[pl.ds(base, CHA), :], bP, gsems[2])

    def process(st, bufs, gsems):
        bS, bT, bP = bufs
        dummy = p_hbm.at[pl.ds(base0, CHA), :]
        pltpu.make_async_copy(dummy, bS, gsems[0]).wait()
        pltpu.make_async_copy(dummy, bT, gsems[1]).wait()
        pltpu.make_async_copy(dummy, bP, gsems[2]).wait()

        def row(r, rc):
            for c8 in range(D // 16):
                sl = pl.ds(c8 * 16, 16)
                x = bS[r, sl] + bT[r, sl] + bP[r, sl]
                bS[r, sl] = _gelu_sc(x)
            return rc

        lax.fori_loop(0, CHA, row, 0)
        pltpu.sync_copy(bS, agg_sh.at[ixt.at[st]], add=True)

    start_idx(0, 0)
    start_idx(1, 1)
    mid(0, 0, bufA, gsemA)
    start_idx(2, 2)
    mid(1, 1, bufB, gsemB)
    start_idx(3, 3)

    def body(k, carry):
        j = 4 * k
        process(0, bufA, gsemA)
        mid(j + 2, 2, bufA, gsemA)
        start_idx(j + 4, 0)
        process(1, bufB, gsemB)
        mid(j + 3, 3, bufB, gsemB)
        start_idx(j + 5, 1)
        process(2, bufA, gsemA)
        mid(j + 4, 0, bufA, gsemA)
        start_idx(j + 6, 2)
        process(3, bufB, gsemB)
        mid(j + 5, 1, bufB, gsemB)
        start_idx(j + 7, 3)
        return carry

    lax.fori_loop(0, NCHA // 4 - 1, body, 0)
    jt = NCHA - 6
    process(0, bufA, gsemA)
    mid(jt + 2, 2, bufA, gsemA)
    start_idx(jt + 4, 0)
    process(1, bufB, gsemB)
    mid(jt + 3, 3, bufB, gsemB)
    start_idx(jt + 5, 1)
    process(2, bufA, gsemA)
    mid(jt + 4, 0, bufA, gsemA)
    process(3, bufB, gsemB)
    mid(jt + 5, 1, bufB, gsemB)
    process(0, bufA, gsemA)
    process(1, bufB, gsemB)
    plsc.subcore_barrier()

    for j in range(RPT // CHA):
        rows = pl.ds(sid * RPT + j * CHA, CHA)
        pltpu.sync_copy(agg_sh.at[rows, :], bufSA)

        @pl.when(cid == 0)
        def _():
            pltpu.sync_copy(bufSA, out0.at[rows, :])

        @pl.when(cid == 1)
        def _():
            pltpu.sync_copy(bufSA, out1.at[rows, :])


@functools.partial(
    pl.kernel,
    out_type=jax.ShapeDtypeStruct((E // 8, 8 * DE), jnp.float32),
    mesh=_MESH,
    compiler_params=pltpu.CompilerParams(use_tc_tiling_on_sc=False),
    scratch_types=[
        pltpu.VMEM((EPW,), jnp.int32),
        pltpu.VMEM((EPW,), jnp.int32),
        pltpu.VMEM((CH, DE), jnp.float32),
        pltpu.VMEM((CH, DE), jnp.float32),
        pltpu.VMEM((CH, DE), jnp.float32),
        pltpu.VMEM((CH, DE), jnp.float32),
        pltpu.VMEM((CH // 8, 8 * DE), jnp.float32),
        pltpu.VMEM((CH // 8, 8 * DE), jnp.float32),
        pltpu.SemaphoreType.DMA,
        pltpu.SemaphoreType.DMA,
        pltpu.SemaphoreType.DMA,
        pltpu.SemaphoreType.DMA,
        pltpu.SemaphoreType.DMA,
    ],
)
def _edge_b(ys_hbm, yt_hbm, src_hbm, tgt_hbm, s_out,
            ixs_all, ixt_all, bufAA, bufBA, bufAB, bufBB, outbA, outbB,
            sAA, sBA, sAB, sBB, sW):
    cid = lax.axis_index("c")
    sid = lax.axis_index("s")
    wid = sid * NC + cid
    base0 = wid * EPW

    pltpu.sync_copy(src_hbm.at[pl.ds(base0, EPW)], ixs_all)
    pltpu.sync_copy(tgt_hbm.at[pl.ds(base0, EPW)], ixt_all)

    def start(i, bA, bB, sA, sB):
        sl = pl.ds(i * CH, CH)
        dA = pltpu.async_copy(ys_hbm.at[ixs_all.at[sl]], bA, sA)
        dB = pltpu.async_copy(yt_hbm.at[ixt_all.at[sl]], bB, sB)
        return (dA, dB)

    def process(i, descs, bA, bB, outb):
        for dsc in descs:
            dsc.wait()

        def prow(q, rc):
            for sub in range(8):
                r = q * 8 + sub
                outb[q, pl.ds(sub * DE, DE)] = (bA[r, pl.ds(0, DE)]
                                                + bB[r, pl.ds(0, DE)])
            return rc

        lax.fori_loop(0, CH // 8, prow, 0)
        rowb = (base0 + i * CH) // 8
        pltpu.async_copy(outb, s_out.at[pl.ds(rowb, CH // 8), :], sW).wait()

    slotA = (bufAA, bufBA)
    slotB = (bufAB, bufBB)
    semsA = (sAA, sBA)
    semsB = (sAB, sBB)
    dA = start(0, *slotA, *semsA)

    def pair(k, carry):
        j = 2 * k
        dB = start(j + 1, *slotB, *semsB)
        process(j, dA, *slotA, outbA)
        start(j + 2, *slotA, *semsA)
        process(j + 1, dB, *slotB, outbB)
        return carry

    lax.fori_loop(0, (NCHUNK - 1) // 2, pair, 0)
    process(NCHUNK - 1, dA, *slotA, outbA)


def kernel(node_features, edge_index, edge_features, msg_W, msg_b,
           W_ih, W_hh, b_ih, b_hh, c1_W, c1_b, c2_W, c2_b):
    src = edge_index[0]
    tgt = edge_index[1]
    nf_pad = jnp.pad(node_features, ((0, NP - N), (0, 0)))

    ef_p = jnp.reshape(edge_features, (E // 8, 8 * DE))

    wst = jnp.concatenate([msg_W[:, :D].T, msg_W[:, D:2 * D].T], axis=1)
    wet = msg_W[:, 2 * D:].T
    bd_we = jax.scipy.linalg.block_diag(*([wet] * 8))
    mb8 = jnp.tile(msg_b, 8).reshape(1, 8 * D)
    wih_t = W_ih.T
    whh_t = W_hh.T
    cst = jnp.concatenate([c1_W[:, :D].T, c1_W[:, D:2 * D].T], axis=1)
    cet = c1_W[:, 2 * D:].T
    c2wt = c2_W.T
    bd_c1 = jax.scipy.linalg.block_diag(*([cet] * 8))
    c1b8 = jnp.tile(c1_b, 8).reshape(1, 8 * DE)
    bd_c2 = jax.scipy.linalg.block_diag(*([c2wt] * 8))
    c2b8 = jnp.tile(c2_b, 8).reshape(1, 8 * NCLS)

    xs, xt = pl.pallas_call(
        _pre_nodes_body,
        grid=(NP // BN,),
        in_specs=[pl.BlockSpec((BN, D), lambda i: (i, 0)),
                  pl.BlockSpec((D, 2 * D), lambda i: (0, 0))],
        out_specs=[pl.BlockSpec((BN, D), lambda i: (i, 0)),
                   pl.BlockSpec((BN, D), lambda i: (i, 0))],
        out_shape=[jax.ShapeDtypeStruct((NP, D), jnp.float32),
                   jax.ShapeDtypeStruct((NP, D), jnp.float32)],
    )(nf_pad, wst)

    be8 = 1000
    p_packed = pl.pallas_call(
        _pre_edges_body,
        grid=(E // 8 // be8,),
        in_specs=[pl.BlockSpec((be8, 8 * DE), lambda i: (i, 0)),
                  pl.BlockSpec((8 * DE, 8 * D), lambda i: (0, 0)),
                  pl.BlockSpec((1, 8 * D), lambda i: (0, 0))],
        out_specs=pl.BlockSpec((be8, 8 * D), lambda i: (i, 0)),
        out_shape=jax.ShapeDtypeStruct((E // 8, 8 * D), jnp.float32),
    )(ef_p, bd_we, mb8)
    p = jnp.reshape(p_packed, (E, D))

    agg0, agg1 = _edge_a(xs, xt, p, src, tgt)

    ys, yt = pl.pallas_call(
        _gru_body,
        grid=(NP // BN,),
        in_specs=[pl.BlockSpec((BN, D), lambda i: (i, 0)),
                  pl.BlockSpec((BN, D), lambda i: (i, 0)),
                  pl.BlockSpec((BN, D), lambda i: (i, 0)),
                  pl.BlockSpec((D, 3 * D), lambda i: (0, 0)),
                  pl.BlockSpec((D, 3 * D), lambda i: (0, 0)),
                  pl.BlockSpec((1, 3 * D), lambda i: (0, 0)),
                  pl.BlockSpec((1, 3 * D), lambda i: (0, 0)),
                  pl.BlockSpec((D, 2 * DE), lambda i: (0, 0))],
        out_specs=[pl.BlockSpec((BN, DE), lambda i: (i, 0)),
                   pl.BlockSpec((BN, DE), lambda i: (i, 0))],
        out_shape=[jax.ShapeDtypeStruct((NP, DE), jnp.float32),
                   jax.ShapeDtypeStruct((NP, DE), jnp.float32)],
    )(agg0, agg1, nf_pad, wih_t, whh_t,
      b_ih.reshape(1, 3 * D), b_hh.reshape(1, 3 * D), cst)

    s = _edge_b(ys, yt, src, tgt)

    lp = pl.pallas_call(
        _cls_body,
        grid=(E // 8 // be8,),
        in_specs=[pl.BlockSpec((be8, 8 * DE), lambda i: (i, 0)),
                  pl.BlockSpec((be8, 8 * DE), lambda i: (i, 0)),
                  pl.BlockSpec((8 * DE, 8 * DE), lambda i: (0, 0)),
                  pl.BlockSpec((1, 8 * DE), lambda i: (0, 0)),
                  pl.BlockSpec((8 * DE, 8 * NCLS), lambda i: (0, 0)),
                  pl.BlockSpec((1, 8 * NCLS), lambda i: (0, 0))],
        out_specs=pl.BlockSpec((be8, 8 * NCLS), lambda i: (i, 0)),
        out_shape=jax.ShapeDtypeStruct((E // 8, 8 * NCLS), jnp.float32),
    )(s, ef_p, bd_c1, c1b8, bd_c2, c2b8)

    return jnp.reshape(lp, (E, NCLS))


# speedup vs baseline: 2.7098x; 1.0213x over previous
import functools
import jax
import jax.numpy as jnp
from jax import lax
from jax.experimental import pallas as pl
from jax.experimental.pallas import tpu as pltpu
from jax.experimental.pallas import tpu_sc as plsc

N = 10000
E = 320000
D = 128
DE = 16
NCLS = 8

NC = 2
NS = 16
NW = NC * NS
EPW = E // NW
CH = 80
NCHUNK = EPW // CH
NP = 10240
RPT = NP // NS
ZR = 128

BN = 2048
BE = 4000

_SQRT_HALF = 0.7071067811865476


def _gelu_sc(x):
    z = jnp.abs(x) * _SQRT_HALF
    t = 1.0 / (z * 0.3275911 + 1.0)
    poly = ((((1.061405429 * t - 1.453152027) * t + 1.421413741) * t
             - 0.284496736) * t + 0.254829592) * t
    erf = jnp.sign(x) * (1.0 - poly * jnp.exp(-(z * z)))
    return 0.5 * x * (1.0 + erf)


def _pre_nodes_body(nf_ref, w_ref, xs_ref, xt_ref):
    d = jnp.dot(nf_ref[...], w_ref[...], preferred_element_type=jnp.float32,
                precision=lax.Precision.HIGHEST)
    xs_ref[...] = d[:, :D]
    xt_ref[...] = d[:, D:]


def _pre_edges_body(efp_ref, w_ref, b_ref, p_ref):
    p_ref[...] = jnp.dot(efp_ref[...], w_ref[...],
                         preferred_element_type=jnp.float32,
                         precision=lax.Precision.HIGHEST) + b_ref[...]


def _gru_body(a0_ref, a1_ref, nf_ref, wih_ref, whh_ref, bih_ref, bhh_ref,
              cst_ref, ys_ref, yt_ref):
    agg = a0_ref[...] + a1_ref[...]
    h = nf_ref[...]
    gi = jnp.dot(agg, wih_ref[...], preferred_element_type=jnp.float32,
                 precision=lax.Precision.HIGHEST) + bih_ref[...]
    gh = jnp.dot(h, whh_ref[...], preferred_element_type=jnp.float32,
                 precision=lax.Precision.HIGHEST) + bhh_ref[...]
    r = jax.nn.sigmoid(gi[:, :D] + gh[:, :D])
    z = jax.nn.sigmoid(gi[:, D:2 * D] + gh[:, D:2 * D])
    n = jnp.tanh(gi[:, 2 * D:] + r * gh[:, 2 * D:])
    hn = (1.0 - z) * n + z * h
    y = jnp.dot(hn, cst_ref[...], preferred_element_type=jnp.float32,
                precision=lax.Precision.HIGHEST)
    ys_ref[...] = y[:, :DE]
    yt_ref[...] = y[:, DE:]


def _cls_body(s_ref, efp_ref, bd1_ref, c1b_ref, bd2_ref, c2b_ref, out_ref):
    x = s_ref[...] + jnp.dot(efp_ref[...], bd1_ref[...],
                             preferred_element_type=jnp.float32,
                             precision=lax.Precision.HIGHEST) + c1b_ref[...]
    hgelu = _gelu_sc(x)
    out_ref[...] = jnp.dot(hgelu, bd2_ref[...],
                           preferred_element_type=jnp.float32,
                           precision=lax.Precision.HIGHEST) + c2b_ref[...]


_MESH = plsc.VectorSubcoreMesh(core_axis_name="c", subcore_axis_name="s",
                               num_cores=NC, num_subcores=NS)


CHA = 40
NCHA = EPW // CHA


@functools.partial(
    pl.kernel,
    out_type=(jax.ShapeDtypeStruct((NP, D), jnp.float32),
              jax.ShapeDtypeStruct((NP, D), jnp.float32)),
    mesh=_MESH,
    compiler_params=pltpu.CompilerParams(use_tc_tiling_on_sc=False),
    scratch_types=[
        pltpu.VMEM((4, CHA), jnp.int32),
        pltpu.VMEM((4, CHA), jnp.int32),
        pltpu.VMEM((CHA, D), jnp.float32),
        pltpu.VMEM((CHA, D), jnp.float32),
        pltpu.VMEM((CHA // 8, 8 * D), jnp.float32),
        pltpu.VMEM((CHA, D), jnp.float32),
        pltpu.VMEM((CHA, D), jnp.float32),
        pltpu.VMEM((CHA // 8, 8 * D), jnp.float32),
        pltpu.VMEM_SHARED((NP, D), jnp.float32),
        pltpu.SemaphoreType.DMA,
        pltpu.SemaphoreType.DMA,
        pltpu.SemaphoreType.DMA,
        pltpu.SemaphoreType.DMA,
        pltpu.SemaphoreType.DMA,
        pltpu.SemaphoreType.DMA,
        pltpu.SemaphoreType.DMA,
        pltpu.SemaphoreType.DMA,
        pltpu.SemaphoreType.DMA,
        pltpu.SemaphoreType.DMA,
        pltpu.SemaphoreType.DMA,
        pltpu.SemaphoreType.DMA,
        pltpu.SemaphoreType.DMA,
        pltpu.SemaphoreType.DMA,
    ],
)
def _edge_a(xs_hbm, xt_hbm, p_hbm, src_hbm, tgt_hbm, out0, out1,
            ixs, ixt,
            bufSA, bufTA, bufPA, bufSB, bufTB, bufPB, agg_sh,
            sI0s, sI0t, sI1s, sI1t, sI2s, sI2t, sI3s, sI3t,
            sSA, sTA, sPA, sSB, sTB, sPB):
    cid = lax.axis_index("c")
    sid = lax.axis_index("s")
    wid = sid * NC + cid
    base0 = wid * EPW

    isems = ((sI0s, sI0t), (sI1s, sI1t), (sI2s, sI2t), (sI3s, sI3t))
    gsemA = (sSA, sTA, sPA)
    gsemB = (sSB, sTB, sPB)
    bufA = (bufSA, bufTA, bufPA)
    bufB = (bufSB, bufTB, bufPB)

    def zrow(r, carry):
        for c8 in range(D // 16):
            bufSA[r, pl.ds(c8 * 16, 16)] = jnp.zeros((16,), jnp.float32)
        return carry

    lax.fori_loop(0, CHA, zrow, 0)
    for j in range(RPT // CHA):
        pltpu.sync_copy(bufSA, agg_sh.at[pl.ds(sid * RPT + j * CHA, CHA), :])
    plsc.subcore_barrier()

    def start_idx(i, st):
        base = base0 + i * CHA
        pltpu.async_copy(src_hbm.at[pl.ds(base, CHA)], ixs.at[st], isems[st][0])
        pltpu.async_copy(tgt_hbm.at[pl.ds(base, CHA)], ixt.at[st], isems[st][1])

    def mid(i, st, bufs, gsems):
        dummy = src_hbm.at[pl.ds(base0, CHA)]
        pltpu.make_async_copy(dummy, ixs.at[st], isems[st][0]).wait()
        pltpu.make_async_copy(dummy, ixt.at[st], isems[st][1]).wait()
        bS, bT, bP = bufs
        pltpu.async_copy(xs_hbm.at[ixs.at[st]], bS, gsems[0])
        pltpu.async_copy(xt_hbm.at[ixt.at[st]], bT, gsems[1])
        prow = (base0 + i * CHA) // 8
        pltpu.async_copy(p_hbm.at[pl.ds(prow, CHA // 8), :], bP, gsems[2])

    def process(st, bufs, gsems):
        bS, bT, bP = bufs
        dummyg = xs_hbm.at[pl.ds(0, CHA), :]
        pltpu.make_async_copy(dummyg, bS, gsems[0]).wait()
        pltpu.make_async_copy(dummyg, bT, gsems[1]).wait()
        pltpu.make_async_copy(p_hbm.at[pl.ds(0, CHA // 8), :], bP,
                              gsems[2]).wait()

        def row(q, rc):
            for sub in range(8):
                r = q * 8 + sub
                for c8 in range(D // 16):
                    sl = pl.ds(c8 * 16, 16)
                    x = (bS[r, sl] + bT[r, sl]
                         + bP[q, pl.ds(sub * D + c8 * 16, 16)])
                    bS[r, sl] = _gelu_sc(x)
            return rc

        lax.fori_loop(0, CHA // 8, row, 0)
        pltpu.sync_copy(bS, agg_sh.at[ixt.at[st]], add=True)

    start_idx(0, 0)
    start_idx(1, 1)
    mid(0, 0, bufA, gsemA)
    start_idx(2, 2)
    mid(1, 1, bufB, gsemB)
    start_idx(3, 3)

    def body(k, carry):
        j = 4 * k
        process(0, bufA, gsemA)
        mid(j + 2, 2, bufA, gsemA)
        start_idx(j + 4, 0)
        process(1, bufB, gsemB)
        mid(j + 3, 3, bufB, gsemB)
        start_idx(j + 5, 1)
        process(2, bufA, gsemA)
        mid(j + 4, 0, bufA, gsemA)
        start_idx(j + 6, 2)
        process(3, bufB, gsemB)
        mid(j + 5, 1, bufB, gsemB)
        start_idx(j + 7, 3)
        return carry

    lax.fori_loop(0, NCHA // 4 - 1, body, 0)
    jt = NCHA - 6
    process(0, bufA, gsemA)
    mid(jt + 2, 2, bufA, gsemA)
    start_idx(jt + 4, 0)
    process(1, bufB, gsemB)
    mid(jt + 3, 3, bufB, gsemB)
    start_idx(jt + 5, 1)
    process(2, bufA, gsemA)
    mid(jt + 4, 0, bufA, gsemA)
    process(3, bufB, gsemB)
    mid(jt + 5, 1, bufB, gsemB)
    process(0, bufA, gsemA)
    process(1, bufB, gsemB)
    plsc.subcore_barrier()

    for j in range(RPT // CHA):
        rows = pl.ds(sid * RPT + j * CHA, CHA)
        pltpu.sync_copy(agg_sh.at[rows, :], bufSA)

        @pl.when(cid == 0)
        def _():
            pltpu.sync_copy(bufSA, out0.at[rows, :])

        @pl.when(cid == 1)
        def _():
            pltpu.sync_copy(bufSA, out1.at[rows, :])


@functools.partial(
    pl.kernel,
    out_type=jax.ShapeDtypeStruct((E // 8, 8 * DE), jnp.float32),
    mesh=_MESH,
    compiler_params=pltpu.CompilerParams(use_tc_tiling_on_sc=False),
    scratch_types=[
        pltpu.VMEM((EPW,), jnp.int32),
        pltpu.VMEM((EPW,), jnp.int32),
        pltpu.VMEM((CH, DE), jnp.float32),
        pltpu.VMEM((CH, DE), jnp.float32),
        pltpu.VMEM((CH, DE), jnp.float32),
        pltpu.VMEM((CH, DE), jnp.float32),
        pltpu.VMEM((CH // 8, 8 * DE), jnp.float32),
        pltpu.VMEM((CH // 8, 8 * DE), jnp.float32),
        pltpu.SemaphoreType.DMA,
        pltpu.SemaphoreType.DMA,
        pltpu.SemaphoreType.DMA,
        pltpu.SemaphoreType.DMA,
        pltpu.SemaphoreType.DMA,
    ],
)
def _edge_b(ys_hbm, yt_hbm, src_hbm, tgt_hbm, s_out,
            ixs_all, ixt_all, bufAA, bufBA, bufAB, bufBB, outbA, outbB,
            sAA, sBA, sAB, sBB, sW):
    cid = lax.axis_index("c")
    sid = lax.axis_index("s")
    wid = sid * NC + cid
    base0 = wid * EPW

    pltpu.sync_copy(src_hbm.at[pl.ds(base0, EPW)], ixs_all)
    pltpu.sync_copy(tgt_hbm.at[pl.ds(base0, EPW)], ixt_all)

    def start(i, bA, bB, sA, sB):
        sl = pl.ds(i * CH, CH)
        dA = pltpu.async_copy(ys_hbm.at[ixs_all.at[sl]], bA, sA)
        dB = pltpu.async_copy(yt_hbm.at[ixt_all.at[sl]], bB, sB)
        return (dA, dB)

    def process(i, descs, bA, bB, outb):
        for dsc in descs:
            dsc.wait()

        def prow(q, rc):
            for sub in range(8):
                r = q * 8 + sub
                outb[q, pl.ds(sub * DE, DE)] = (bA[r, pl.ds(0, DE)]
                                                + bB[r, pl.ds(0, DE)])
            return rc

        lax.fori_loop(0, CH // 8, prow, 0)
        rowb = (base0 + i * CH) // 8
        pltpu.async_copy(outb, s_out.at[pl.ds(rowb, CH // 8), :], sW).wait()

    slotA = (bufAA, bufBA)
    slotB = (bufAB, bufBB)
    semsA = (sAA, sBA)
    semsB = (sAB, sBB)
    dA = start(0, *slotA, *semsA)

    def pair(k, carry):
        j = 2 * k
        dB = start(j + 1, *slotB, *semsB)
        process(j, dA, *slotA, outbA)
        start(j + 2, *slotA, *semsA)
        process(j + 1, dB, *slotB, outbB)
        return carry

    lax.fori_loop(0, (NCHUNK - 1) // 2, pair, 0)
    process(NCHUNK - 1, dA, *slotA, outbA)


def kernel(node_features, edge_index, edge_features, msg_W, msg_b,
           W_ih, W_hh, b_ih, b_hh, c1_W, c1_b, c2_W, c2_b):
    src = edge_index[0]
    tgt = edge_index[1]
    nf_pad = jnp.pad(node_features, ((0, NP - N), (0, 0)))

    ef_p = jnp.reshape(edge_features, (E // 8, 8 * DE))

    wst = jnp.concatenate([msg_W[:, :D].T, msg_W[:, D:2 * D].T], axis=1)
    wet = msg_W[:, 2 * D:].T
    bd_we = jax.scipy.linalg.block_diag(*([wet] * 8))
    mb8 = jnp.tile(msg_b, 8).reshape(1, 8 * D)
    wih_t = W_ih.T
    whh_t = W_hh.T
    cst = jnp.concatenate([c1_W[:, :D].T, c1_W[:, D:2 * D].T], axis=1)
    cet = c1_W[:, 2 * D:].T
    c2wt = c2_W.T
    bd_c1 = jax.scipy.linalg.block_diag(*([cet] * 8))
    c1b8 = jnp.tile(c1_b, 8).reshape(1, 8 * DE)
    bd_c2 = jax.scipy.linalg.block_diag(*([c2wt] * 8))
    c2b8 = jnp.tile(c2_b, 8).reshape(1, 8 * NCLS)

    xs, xt = pl.pallas_call(
        _pre_nodes_body,
        grid=(NP // BN,),
        in_specs=[pl.BlockSpec((BN, D), lambda i: (i, 0)),
                  pl.BlockSpec((D, 2 * D), lambda i: (0, 0))],
        out_specs=[pl.BlockSpec((BN, D), lambda i: (i, 0)),
                   pl.BlockSpec((BN, D), lambda i: (i, 0))],
        out_shape=[jax.ShapeDtypeStruct((NP, D), jnp.float32),
                   jax.ShapeDtypeStruct((NP, D), jnp.float32)],
    )(nf_pad, wst)

    be8 = 1000
    p_packed = pl.pallas_call(
        _pre_edges_body,
        grid=(E // 8 // be8,),
        in_specs=[pl.BlockSpec((be8, 8 * DE), lambda i: (i, 0)),
                  pl.BlockSpec((8 * DE, 8 * D), lambda i: (0, 0)),
                  pl.BlockSpec((1, 8 * D), lambda i: (0, 0))],
        out_specs=pl.BlockSpec((be8, 8 * D), lambda i: (i, 0)),
        out_shape=jax.ShapeDtypeStruct((E // 8, 8 * D), jnp.float32),
    )(ef_p, bd_we, mb8)

    agg0, agg1 = _edge_a(xs, xt, p_packed, src, tgt)

    ys, yt = pl.pallas_call(
        _gru_body,
        grid=(NP // BN,),
        in_specs=[pl.BlockSpec((BN, D), lambda i: (i, 0)),
                  pl.BlockSpec((BN, D), lambda i: (i, 0)),
                  pl.BlockSpec((BN, D), lambda i: (i, 0)),
                  pl.BlockSpec((D, 3 * D), lambda i: (0, 0)),
                  pl.BlockSpec((D, 3 * D), lambda i: (0, 0)),
                  pl.BlockSpec((1, 3 * D), lambda i: (0, 0)),
                  pl.BlockSpec((1, 3 * D), lambda i: (0, 0)),
                  pl.BlockSpec((D, 2 * DE), lambda i: (0, 0))],
        out_specs=[pl.BlockSpec((BN, DE), lambda i: (i, 0)),
                   pl.BlockSpec((BN, DE), lambda i: (i, 0))],
        out_shape=[jax.ShapeDtypeStruct((NP, DE), jnp.float32),
                   jax.ShapeDtypeStruct((NP, DE), jnp.float32)],
    )(agg0, agg1, nf_pad, wih_t, whh_t,
      b_ih.reshape(1, 3 * D), b_hh.reshape(1, 3 * D), cst)

    s = _edge_b(ys, yt, src, tgt)

    lp = pl.pallas_call(
        _cls_body,
        grid=(E // 8 // be8,),
        in_specs=[pl.BlockSpec((be8, 8 * DE), lambda i: (i, 0)),
                  pl.BlockSpec((be8, 8 * DE), lambda i: (i, 0)),
                  pl.BlockSpec((8 * DE, 8 * DE), lambda i: (0, 0)),
                  pl.BlockSpec((1, 8 * DE), lambda i: (0, 0)),
                  pl.BlockSpec((8 * DE, 8 * NCLS), lambda i: (0, 0)),
                  pl.BlockSpec((1, 8 * NCLS), lambda i: (0, 0))],
        out_specs=pl.BlockSpec((be8, 8 * NCLS), lambda i: (i, 0)),
        out_shape=jax.ShapeDtypeStruct((E // 8, 8 * NCLS), jnp.float32),
    )(s, ef_p, bd_c1, c1b8, bd_c2, c2b8)

    return jnp.reshape(lp, (E, NCLS))


# speedup vs baseline: 3.1006x; 1.1442x over previous
import functools
import jax
import jax.numpy as jnp
from jax import lax
from jax.experimental import pallas as pl
from jax.experimental.pallas import tpu as pltpu
from jax.experimental.pallas import tpu_sc as plsc

N = 10000
E = 320000
D = 128
DE = 16
NCLS = 8

NC = 2
NS = 16
NW = NC * NS
EPW = E // NW
CH = 80
NCHUNK = EPW // CH
NP = 10240
RPT = NP // NS
ZR = 128

BN = 2048
BE = 4000

_SQRT_HALF = 0.7071067811865476


def _gelu_sc(x):
    z = jnp.abs(x) * _SQRT_HALF
    t = 1.0 / (z * 0.3275911 + 1.0)
    poly = ((((1.061405429 * t - 1.453152027) * t + 1.421413741) * t
             - 0.284496736) * t + 0.254829592) * t
    erf = jnp.sign(x) * (1.0 - poly * jnp.exp(-(z * z)))
    return 0.5 * x * (1.0 + erf)


def _pre_nodes_body(nf_ref, w_ref, xs_ref, xt_ref):
    d = jnp.dot(nf_ref[...], w_ref[...], preferred_element_type=jnp.float32,
                precision=lax.Precision.HIGHEST)
    xs_ref[...] = d[:, :D]
    xt_ref[...] = d[:, D:]


def _pre_edges_body(eft_ref, w_ref, b_ref, p_ref):
    p_ref[...] = lax.dot_general(
        eft_ref[...], w_ref[...], (((0,), (0,)), ((), ())),
        preferred_element_type=jnp.float32,
        precision=lax.Precision.HIGHEST) + b_ref[...]


def _gru_body(a0_ref, a1_ref, nf_ref, wih_ref, whh_ref, bih_ref, bhh_ref,
              cst_ref, ys_ref, yt_ref):
    agg = a0_ref[...] + a1_ref[...]
    h = nf_ref[...]
    gi = jnp.dot(agg, wih_ref[...], preferred_element_type=jnp.float32,
                 precision=lax.Precision.HIGHEST) + bih_ref[...]
    gh = jnp.dot(h, whh_ref[...], preferred_element_type=jnp.float32,
                 precision=lax.Precision.HIGHEST) + bhh_ref[...]
    r = jax.nn.sigmoid(gi[:, :D] + gh[:, :D])
    z = jax.nn.sigmoid(gi[:, D:2 * D] + gh[:, D:2 * D])
    n = jnp.tanh(gi[:, 2 * D:] + r * gh[:, 2 * D:])
    hn = (1.0 - z) * n + z * h
    y = jnp.dot(hn, cst_ref[...], preferred_element_type=jnp.float32,
                precision=lax.Precision.HIGHEST)
    ys_ref[...] = y[:, :DE]
    yt_ref[...] = y[:, DE:]


def _cls_body(s_ref, efp_ref, bd1_ref, c1b_ref, bd2_ref, c2b_ref, out_ref):
    x = s_ref[...] + jnp.dot(efp_ref[...], bd1_ref[...],
                             preferred_element_type=jnp.float32,
                             precision=lax.Precision.HIGHEST) + c1b_ref[...]
    hgelu = _gelu_sc(x)
    out_ref[...] = jnp.dot(hgelu, bd2_ref[...],
                           preferred_element_type=jnp.float32,
                           precision=lax.Precision.HIGHEST) + c2b_ref[...]


_MESH = plsc.VectorSubcoreMesh(core_axis_name="c", subcore_axis_name="s",
                               num_cores=NC, num_subcores=NS)


CHA = 40
NCHA = EPW // CHA


@functools.partial(
    pl.kernel,
    out_type=(jax.ShapeDtypeStruct((NP, D), jnp.float32),
              jax.ShapeDtypeStruct((NP, D), jnp.float32)),
    mesh=_MESH,
    scratch_types=[
        pltpu.VMEM((4, CHA), jnp.int32),
        pltpu.VMEM((4, CHA), jnp.int32),
        pltpu.VMEM((CHA, D), jnp.float32),
        pltpu.VMEM((CHA, D), jnp.float32),
        pltpu.VMEM((CHA, D), jnp.float32),
        pltpu.VMEM((CHA, D), jnp.float32),
        pltpu.VMEM((CHA, D), jnp.float32),
        pltpu.VMEM((CHA, D), jnp.float32),
        pltpu.VMEM_SHARED((NP, D), jnp.float32),
        pltpu.SemaphoreType.DMA,
        pltpu.SemaphoreType.DMA,
        pltpu.SemaphoreType.DMA,
        pltpu.SemaphoreType.DMA,
        pltpu.SemaphoreType.DMA,
        pltpu.SemaphoreType.DMA,
        pltpu.SemaphoreType.DMA,
        pltpu.SemaphoreType.DMA,
        pltpu.SemaphoreType.DMA,
        pltpu.SemaphoreType.DMA,
        pltpu.SemaphoreType.DMA,
        pltpu.SemaphoreType.DMA,
        pltpu.SemaphoreType.DMA,
        pltpu.SemaphoreType.DMA,
    ],
)
def _edge_a(xs_hbm, xt_hbm, p_hbm, src_hbm, tgt_hbm, out0, out1,
            ixs, ixt,
            bufSA, bufTA, bufPA, bufSB, bufTB, bufPB, agg_sh,
            sI0s, sI0t, sI1s, sI1t, sI2s, sI2t, sI3s, sI3t,
            sSA, sTA, sPA, sSB, sTB, sPB):
    cid = lax.axis_index("c")
    sid = lax.axis_index("s")
    wid = sid * NC + cid
    base0 = wid * EPW

    isems = ((sI0s, sI0t), (sI1s, sI1t), (sI2s, sI2t), (sI3s, sI3t))
    gsemA = (sSA, sTA, sPA)
    gsemB = (sSB, sTB, sPB)
    bufA = (bufSA, bufTA, bufPA)
    bufB = (bufSB, bufTB, bufPB)

    def zrow(r, carry):
        for c8 in range(D // 16):
            bufSA[r, pl.ds(c8 * 16, 16)] = jnp.zeros((16,), jnp.float32)
        return carry

    lax.fori_loop(0, CHA, zrow, 0)
    for j in range(RPT // CHA):
        pltpu.sync_copy(bufSA, agg_sh.at[pl.ds(sid * RPT + j * CHA, CHA), :])
    plsc.subcore_barrier()

    def start_idx(i, st):
        base = base0 + i * CHA
        pltpu.async_copy(src_hbm.at[pl.ds(base, CHA)], ixs.at[st], isems[st][0])
        pltpu.async_copy(tgt_hbm.at[pl.ds(base, CHA)], ixt.at[st], isems[st][1])

    def mid(i, st, bufs, gsems):
        dummy = src_hbm.at[pl.ds(base0, CHA)]
        pltpu.make_async_copy(dummy, ixs.at[st], isems[st][0]).wait()
        pltpu.make_async_copy(dummy, ixt.at[st], isems[st][1]).wait()
        base = base0 + i * CHA
        bS, bT, bP = bufs
        pltpu.async_copy(xs_hbm.at[ixs.at[st]], bS, gsems[0])
        pltpu.async_copy(xt_hbm.at[ixt.at[st]], bT, gsems[1])
        pltpu.async_copy(p_hbm.at[pl.ds(base, CHA), :], bP, gsems[2])

    def process(st, bufs, gsems):
        bS, bT, bP = bufs
        dummyg = xs_hbm.at[pl.ds(0, CHA), :]
        pltpu.make_async_copy(dummyg, bS, gsems[0]).wait()
        pltpu.make_async_copy(dummyg, bT, gsems[1]).wait()
        pltpu.make_async_copy(dummyg, bP, gsems[2]).wait()

        def row(r, rc):
            for c8 in range(D // 16):
                sl = pl.ds(c8 * 16, 16)
                x = bS[r, sl] + bT[r, sl] + bP[r, sl]
                bS[r, sl] = _gelu_sc(x)
            return rc

        lax.fori_loop(0, CHA, row, 0)
        pltpu.sync_copy(bS, agg_sh.at[ixt.at[st]], add=True)

    start_idx(0, 0)
    start_idx(1, 1)
    mid(0, 0, bufA, gsemA)
    start_idx(2, 2)
    mid(1, 1, bufB, gsemB)
    start_idx(3, 3)

    def body(k, carry):
        j = 4 * k
        process(0, bufA, gsemA)
        mid(j + 2, 2, bufA, gsemA)
        start_idx(j + 4, 0)
        process(1, bufB, gsemB)
        mid(j + 3, 3, bufB, gsemB)
        start_idx(j + 5, 1)
        process(2, bufA, gsemA)
        mid(j + 4, 0, bufA, gsemA)
        start_idx(j + 6, 2)
        process(3, bufB, gsemB)
        mid(j + 5, 1, bufB, gsemB)
        start_idx(j + 7, 3)
        return carry

    lax.fori_loop(0, NCHA // 4 - 1, body, 0)
    jt = NCHA - 6
    process(0, bufA, gsemA)
    mid(jt + 2, 2, bufA, gsemA)
    start_idx(jt + 4, 0)
    process(1, bufB, gsemB)
    mid(jt + 3, 3, bufB, gsemB)
    start_idx(jt + 5, 1)
    process(2, bufA, gsemA)
    mid(jt + 4, 0, bufA, gsemA)
    process(3, bufB, gsemB)
    mid(jt + 5, 1, bufB, gsemB)
    process(0, bufA, gsemA)
    process(1, bufB, gsemB)
    plsc.subcore_barrier()

    for j in range(RPT // CHA):
        rows = pl.ds(sid * RPT + j * CHA, CHA)
        pltpu.sync_copy(agg_sh.at[rows, :], bufSA)

        @pl.when(cid == 0)
        def _():
            pltpu.sync_copy(bufSA, out0.at[rows, :])

        @pl.when(cid == 1)
        def _():
            pltpu.sync_copy(bufSA, out1.at[rows, :])


@functools.partial(
    pl.kernel,
    out_type=jax.ShapeDtypeStruct((E // 8, 8 * DE), jnp.float32),
    mesh=_MESH,
    compiler_params=pltpu.CompilerParams(use_tc_tiling_on_sc=False),
    scratch_types=[
        pltpu.VMEM((EPW,), jnp.int32),
        pltpu.VMEM((EPW,), jnp.int32),
        pltpu.VMEM((CH, DE), jnp.float32),
        pltpu.VMEM((CH, DE), jnp.float32),
        pltpu.VMEM((CH, DE), jnp.float32),
        pltpu.VMEM((CH, DE), jnp.float32),
        pltpu.VMEM((CH // 8, 8 * DE), jnp.float32),
        pltpu.VMEM((CH // 8, 8 * DE), jnp.float32),
        pltpu.SemaphoreType.DMA,
        pltpu.SemaphoreType.DMA,
        pltpu.SemaphoreType.DMA,
        pltpu.SemaphoreType.DMA,
        pltpu.SemaphoreType.DMA,
    ],
)
def _edge_b(ys_hbm, yt_hbm, src_hbm, tgt_hbm, s_out,
            ixs_all, ixt_all, bufAA, bufBA, bufAB, bufBB, outbA, outbB,
            sAA, sBA, sAB, sBB, sW):
    cid = lax.axis_index("c")
    sid = lax.axis_index("s")
    wid = sid * NC + cid
    base0 = wid * EPW

    pltpu.sync_copy(src_hbm.at[pl.ds(base0, EPW)], ixs_all)
    pltpu.sync_copy(tgt_hbm.at[pl.ds(base0, EPW)], ixt_all)

    def start(i, bA, bB, sA, sB):
        sl = pl.ds(i * CH, CH)
        dA = pltpu.async_copy(ys_hbm.at[ixs_all.at[sl]], bA, sA)
        dB = pltpu.async_copy(yt_hbm.at[ixt_all.at[sl]], bB, sB)
        return (dA, dB)

    def process(i, descs, bA, bB, outb):
        for dsc in descs:
            dsc.wait()

        def prow(q, rc):
            for sub in range(8):
                r = q * 8 + sub
                outb[q, pl.ds(sub * DE, DE)] = (bA[r, pl.ds(0, DE)]
                                                + bB[r, pl.ds(0, DE)])
            return rc

        lax.fori_loop(0, CH // 8, prow, 0)
        rowb = (base0 + i * CH) // 8
        pltpu.async_copy(outb, s_out.at[pl.ds(rowb, CH // 8), :], sW).wait()

    slotA = (bufAA, bufBA)
    slotB = (bufAB, bufBB)
    semsA = (sAA, sBA)
    semsB = (sAB, sBB)
    dA = start(0, *slotA, *semsA)

    def pair(k, carry):
        j = 2 * k
        dB = start(j + 1, *slotB, *semsB)
        process(j, dA, *slotA, outbA)
        start(j + 2, *slotA, *semsA)
        process(j + 1, dB, *slotB, outbB)
        return carry

    lax.fori_loop(0, (NCHUNK - 1) // 2, pair, 0)
    process(NCHUNK - 1, dA, *slotA, outbA)


def kernel(node_features, edge_index, edge_features, msg_W, msg_b,
           W_ih, W_hh, b_ih, b_hh, c1_W, c1_b, c2_W, c2_b):
    src = edge_index[0]
    tgt = edge_index[1]
    nf_pad = jnp.pad(node_features, ((0, NP - N), (0, 0)))

    ef_p = jnp.reshape(edge_features, (E // 8, 8 * DE))

    wst = jnp.concatenate([msg_W[:, :D].T, msg_W[:, D:2 * D].T], axis=1)
    wet = msg_W[:, 2 * D:].T
    bd_we = jax.scipy.linalg.block_diag(*([wet] * 8))
    mb8 = jnp.tile(msg_b, 8).reshape(1, 8 * D)
    wih_t = W_ih.T
    whh_t = W_hh.T
    cst = jnp.concatenate([c1_W[:, :D].T, c1_W[:, D:2 * D].T], axis=1)
    cet = c1_W[:, 2 * D:].T
    c2wt = c2_W.T
    bd_c1 = jax.scipy.linalg.block_diag(*([cet] * 8))
    c1b8 = jnp.tile(c1_b, 8).reshape(1, 8 * DE)
    bd_c2 = jax.scipy.linalg.block_diag(*([c2wt] * 8))
    c2b8 = jnp.tile(c2_b, 8).reshape(1, 8 * NCLS)

    xs, xt = pl.pallas_call(
        _pre_nodes_body,
        grid=(NP // BN,),
        in_specs=[pl.BlockSpec((BN, D), lambda i: (i, 0)),
                  pl.BlockSpec((D, 2 * D), lambda i: (0, 0))],
        out_specs=[pl.BlockSpec((BN, D), lambda i: (i, 0)),
                   pl.BlockSpec((BN, D), lambda i: (i, 0))],
        out_shape=[jax.ShapeDtypeStruct((NP, D), jnp.float32),
                   jax.ShapeDtypeStruct((NP, D), jnp.float32)],
    )(nf_pad, wst)

    be8 = 1000
    eft = edge_features.T
    bep = 6400
    p = pl.pallas_call(
        _pre_edges_body,
        grid=(E // bep,),
        in_specs=[pl.BlockSpec((DE, bep), lambda i: (0, i)),
                  pl.BlockSpec((DE, D), lambda i: (0, 0)),
                  pl.BlockSpec((1, D), lambda i: (0, 0))],
        out_specs=pl.BlockSpec((bep, D), lambda i: (i, 0)),
        out_shape=jax.ShapeDtypeStruct((E, D), jnp.float32),
        compiler_params=pltpu.CompilerParams(
            fuse_transposed_lhs_in_matmul=True),
    )(eft, wet, msg_b.reshape(1, D))

    agg0, agg1 = _edge_a(xs, xt, p, src, tgt)

    ys, yt = pl.pallas_call(
        _gru_body,
        grid=(NP // BN,),
        in_specs=[pl.BlockSpec((BN, D), lambda i: (i, 0)),
                  pl.BlockSpec((BN, D), lambda i: (i, 0)),
                  pl.BlockSpec((BN, D), lambda i: (i, 0)),
                  pl.BlockSpec((D, 3 * D), lambda i: (0, 0)),
                  pl.BlockSpec((D, 3 * D), lambda i: (0, 0)),
                  pl.BlockSpec((1, 3 * D), lambda i: (0, 0)),
                  pl.BlockSpec((1, 3 * D), lambda i: (0, 0)),
                  pl.BlockSpec((D, 2 * DE), lambda i: (0, 0))],
        out_specs=[pl.BlockSpec((BN, DE), lambda i: (i, 0)),
                   pl.BlockSpec((BN, DE), lambda i: (i, 0))],
        out_shape=[jax.ShapeDtypeStruct((NP, DE), jnp.float32),
                   jax.ShapeDtypeStruct((NP, DE), jnp.float32)],
    )(agg0, agg1, nf_pad, wih_t, whh_t,
      b_ih.reshape(1, 3 * D), b_hh.reshape(1, 3 * D), cst)

    s = _edge_b(ys, yt, src, tgt)

    lp = pl.pallas_call(
        _cls_body,
        grid=(E // 8 // be8,),
        in_specs=[pl.BlockSpec((be8, 8 * DE), lambda i: (i, 0)),
                  pl.BlockSpec((be8, 8 * DE), lambda i: (i, 0)),
                  pl.BlockSpec((8 * DE, 8 * DE), lambda i: (0, 0)),
                  pl.BlockSpec((1, 8 * DE), lambda i: (0, 0)),
                  pl.BlockSpec((8 * DE, 8 * NCLS), lambda i: (0, 0)),
                  pl.BlockSpec((1, 8 * NCLS), lambda i: (0, 0))],
        out_specs=pl.BlockSpec((be8, 8 * NCLS), lambda i: (i, 0)),
        out_shape=jax.ShapeDtypeStruct((E // 8, 8 * NCLS), jnp.float32),
    )(s, ef_p, bd_c1, c1b8, bd_c2, c2b8)

    return jnp.reshape(lp, (E, NCLS))


# speedup vs baseline: 3.1376x; 1.0119x over previous
import functools
import jax
import jax.numpy as jnp
from jax import lax
from jax.experimental import pallas as pl
from jax.experimental.pallas import tpu as pltpu
from jax.experimental.pallas import tpu_sc as plsc

N = 10000
E = 320000
D = 128
DE = 16
NCLS = 8

NC = 2
NS = 16
NW = NC * NS
EPW = E // NW
CH = 80
NCHUNK = EPW // CH
NP = 10240
RPT = NP // NS
ZR = 128

BN = 2048
BE = 4000

_SQRT_HALF = 0.7071067811865476


def _gelu_sc(x):
    z = jnp.abs(x) * _SQRT_HALF
    t = 1.0 / (z * 0.3275911 + 1.0)
    poly = ((((1.061405429 * t - 1.453152027) * t + 1.421413741) * t
             - 0.284496736) * t + 0.254829592) * t
    erf = jnp.sign(x) * (1.0 - poly * jnp.exp(-(z * z)))
    return 0.5 * x * (1.0 + erf)


def _pre_nodes_body(nf_ref, w_ref, xs_ref, xt_ref):
    d = jnp.dot(nf_ref[...], w_ref[...], preferred_element_type=jnp.float32,
                precision=lax.Precision.HIGHEST)
    xs_ref[...] = d[:, :D]
    xt_ref[...] = d[:, D:]


def _pre_edges_body(eft_ref, w_ref, b_ref, p_ref):
    p_ref[...] = lax.dot_general(
        eft_ref[...], w_ref[...], (((0,), (0,)), ((), ())),
        preferred_element_type=jnp.float32,
        precision=lax.Precision.HIGHEST) + b_ref[...]


def _gru_body(a0_ref, a1_ref, nf_ref, wih_ref, whh_ref, bih_ref, bhh_ref,
              cst_ref, ys_ref, yt_ref):
    agg = a0_ref[...] + a1_ref[...]
    h = nf_ref[...]
    gi = jnp.dot(agg, wih_ref[...], preferred_element_type=jnp.float32,
                 precision=lax.Precision.HIGHEST) + bih_ref[...]
    gh = jnp.dot(h, whh_ref[...], preferred_element_type=jnp.float32,
                 precision=lax.Precision.HIGHEST) + bhh_ref[...]
    r = jax.nn.sigmoid(gi[:, :D] + gh[:, :D])
    z = jax.nn.sigmoid(gi[:, D:2 * D] + gh[:, D:2 * D])
    n = jnp.tanh(gi[:, 2 * D:] + r * gh[:, 2 * D:])
    hn = (1.0 - z) * n + z * h
    y = jnp.dot(hn, cst_ref[...], preferred_element_type=jnp.float32,
                precision=lax.Precision.HIGHEST)
    ys_ref[...] = y[:, :DE]
    yt_ref[...] = y[:, DE:]


def _cls_body(s_ref, efp_ref, bd1_ref, c1b_ref, bd2_ref, c2b_ref, out_ref):
    x = s_ref[...] + jnp.dot(efp_ref[...], bd1_ref[...],
                             preferred_element_type=jnp.float32,
                             precision=lax.Precision.HIGHEST) + c1b_ref[...]
    hgelu = _gelu_sc(x)
    out_ref[...] = jnp.dot(hgelu, bd2_ref[...],
                           preferred_element_type=jnp.float32,
                           precision=lax.Precision.HIGHEST) + c2b_ref[...]


_MESH = plsc.VectorSubcoreMesh(core_axis_name="c", subcore_axis_name="s",
                               num_cores=NC, num_subcores=NS)


CHA = 40
NCHA = EPW // CHA


@functools.partial(
    pl.kernel,
    out_type=(jax.ShapeDtypeStruct((NP, D), jnp.float32),
              jax.ShapeDtypeStruct((NP, D), jnp.float32)),
    mesh=_MESH,
    scratch_types=[
        pltpu.VMEM((4, CHA), jnp.int32),
        pltpu.VMEM((4, CHA), jnp.int32),
        pltpu.VMEM((CHA, D), jnp.float32),
        pltpu.VMEM((CHA, D), jnp.float32),
        pltpu.VMEM((CHA, D), jnp.float32),
        pltpu.VMEM((CHA, D), jnp.float32),
        pltpu.VMEM((CHA, D), jnp.float32),
        pltpu.VMEM((CHA, D), jnp.float32),
        pltpu.VMEM_SHARED((NP, D), jnp.float32),
        pltpu.SemaphoreType.DMA,
        pltpu.SemaphoreType.DMA,
        pltpu.SemaphoreType.DMA,
        pltpu.SemaphoreType.DMA,
        pltpu.SemaphoreType.DMA,
        pltpu.SemaphoreType.DMA,
        pltpu.SemaphoreType.DMA,
        pltpu.SemaphoreType.DMA,
        pltpu.SemaphoreType.DMA,
        pltpu.SemaphoreType.DMA,
        pltpu.SemaphoreType.DMA,
        pltpu.SemaphoreType.DMA,
        pltpu.SemaphoreType.DMA,
        pltpu.SemaphoreType.DMA,
    ],
)
def _edge_a(xs_hbm, xt_hbm, p_hbm, src_hbm, tgt_hbm, out0, out1,
            ixs, ixt,
            bufSA, bufTA, bufPA, bufSB, bufTB, bufPB, agg_sh,
            sI0s, sI0t, sI1s, sI1t, sI2s, sI2t, sI3s, sI3t,
            sSA, sTA, sPA, sSB, sTB, sPB):
    cid = lax.axis_index("c")
    sid = lax.axis_index("s")
    wid = sid * NC + cid
    base0 = wid * EPW

    isems = ((sI0s, sI0t), (sI1s, sI1t), (sI2s, sI2t), (sI3s, sI3t))
    gsemA = (sSA, sTA, sPA)
    gsemB = (sSB, sTB, sPB)
    bufA = (bufSA, bufTA, bufPA)
    bufB = (bufSB, bufTB, bufPB)

    def zrow(r, carry):
        for c8 in range(D // 16):
            bufSA[r, pl.ds(c8 * 16, 16)] = jnp.zeros((16,), jnp.float32)
        return carry

    lax.fori_loop(0, CHA, zrow, 0)
    for j in range(RPT // CHA):
        pltpu.sync_copy(bufSA, agg_sh.at[pl.ds(sid * RPT + j * CHA, CHA), :])
    plsc.subcore_barrier()

    def start_idx(i, st):
        base = base0 + i * CHA
        pltpu.async_copy(src_hbm.at[pl.ds(base, CHA)], ixs.at[st], isems[st][0])
        pltpu.async_copy(tgt_hbm.at[pl.ds(base, CHA)], ixt.at[st], isems[st][1])

    def mid(i, st, bufs, gsems):
        dummy = src_hbm.at[pl.ds(base0, CHA)]
        pltpu.make_async_copy(dummy, ixs.at[st], isems[st][0]).wait()
        pltpu.make_async_copy(dummy, ixt.at[st], isems[st][1]).wait()
        base = base0 + i * CHA
        bS, bT, bP = bufs
        pltpu.async_copy(xs_hbm.at[ixs.at[st]], bS, gsems[0])
        pltpu.async_copy(xt_hbm.at[ixt.at[st]], bT, gsems[1])
        pltpu.async_copy(p_hbm.at[pl.ds(base, CHA), :], bP, gsems[2])

    def process(st, bufs, gsems):
        bS, bT, bP = bufs
        dummyg = xs_hbm.at[pl.ds(0, CHA), :]
        pltpu.make_async_copy(dummyg, bS, gsems[0]).wait()
        pltpu.make_async_copy(dummyg, bT, gsems[1]).wait()
        pltpu.make_async_copy(dummyg, bP, gsems[2]).wait()

        def row(r, rc):
            for c8 in range(D // 16):
                sl = pl.ds(c8 * 16, 16)
                x = bS[r, sl] + bT[r, sl] + bP[r, sl]
                bS[r, sl] = _gelu_sc(x)
            return rc

        lax.fori_loop(0, CHA, row, 0)
        pltpu.sync_copy(bS, agg_sh.at[ixt.at[st]], add=True)

    start_idx(0, 0)
    start_idx(1, 1)
    mid(0, 0, bufA, gsemA)
    start_idx(2, 2)
    mid(1, 1, bufB, gsemB)
    start_idx(3, 3)

    def body(k, carry):
        j = 4 * k
        process(0, bufA, gsemA)
        mid(j + 2, 2, bufA, gsemA)
        start_idx(j + 4, 0)
        process(1, bufB, gsemB)
        mid(j + 3, 3, bufB, gsemB)
        start_idx(j + 5, 1)
        process(2, bufA, gsemA)
        mid(j + 4, 0, bufA, gsemA)
        start_idx(j + 6, 2)
        process(3, bufB, gsemB)
        mid(j + 5, 1, bufB, gsemB)
        start_idx(j + 7, 3)
        return carry

    lax.fori_loop(0, NCHA // 4 - 1, body, 0)
    jt = NCHA - 6
    process(0, bufA, gsemA)
    mid(jt + 2, 2, bufA, gsemA)
    start_idx(jt + 4, 0)
    process(1, bufB, gsemB)
    mid(jt + 3, 3, bufB, gsemB)
    start_idx(jt + 5, 1)
    process(2, bufA, gsemA)
    mid(jt + 4, 0, bufA, gsemA)
    process(3, bufB, gsemB)
    mid(jt + 5, 1, bufB, gsemB)
    process(0, bufA, gsemA)
    process(1, bufB, gsemB)
    plsc.subcore_barrier()

    for j in range(RPT // CHA):
        rows = pl.ds(sid * RPT + j * CHA, CHA)
        pltpu.sync_copy(agg_sh.at[rows, :], bufSA)

        @pl.when(cid == 0)
        def _():
            pltpu.sync_copy(bufSA, out0.at[rows, :])

        @pl.when(cid == 1)
        def _():
            pltpu.sync_copy(bufSA, out1.at[rows, :])


@functools.partial(
    pl.kernel,
    out_type=jax.ShapeDtypeStruct((E // 16, 16 * DE), jnp.float32),
    mesh=_MESH,
    compiler_params=pltpu.CompilerParams(use_tc_tiling_on_sc=False),
    scratch_types=[
        pltpu.VMEM((EPW,), jnp.int32),
        pltpu.VMEM((EPW,), jnp.int32),
        pltpu.VMEM((CH, DE), jnp.float32),
        pltpu.VMEM((CH, DE), jnp.float32),
        pltpu.VMEM((CH, DE), jnp.float32),
        pltpu.VMEM((CH, DE), jnp.float32),
        pltpu.VMEM((CH // 16, 16 * DE), jnp.float32),
        pltpu.VMEM((CH // 16, 16 * DE), jnp.float32),
        pltpu.SemaphoreType.DMA,
        pltpu.SemaphoreType.DMA,
        pltpu.SemaphoreType.DMA,
        pltpu.SemaphoreType.DMA,
        pltpu.SemaphoreType.DMA,
    ],
)
def _edge_b(ys_hbm, yt_hbm, src_hbm, tgt_hbm, s_out,
            ixs_all, ixt_all, bufAA, bufBA, bufAB, bufBB, outbA, outbB,
            sAA, sBA, sAB, sBB, sW):
    cid = lax.axis_index("c")
    sid = lax.axis_index("s")
    wid = sid * NC + cid
    base0 = wid * EPW

    pltpu.sync_copy(src_hbm.at[pl.ds(base0, EPW)], ixs_all)
    pltpu.sync_copy(tgt_hbm.at[pl.ds(base0, EPW)], ixt_all)

    def start(i, bA, bB, sA, sB):
        sl = pl.ds(i * CH, CH)
        dA = pltpu.async_copy(ys_hbm.at[ixs_all.at[sl]], bA, sA)
        dB = pltpu.async_copy(yt_hbm.at[ixt_all.at[sl]], bB, sB)
        return (dA, dB)

    def process(i, descs, bA, bB, outb):
        for dsc in descs:
            dsc.wait()

        def prow(q, rc):
            for sub in range(16):
                r = q * 16 + sub
                outb[q, pl.ds(sub * DE, DE)] = (bA[r, pl.ds(0, DE)]
                                                + bB[r, pl.ds(0, DE)])
            return rc

        lax.fori_loop(0, CH // 16, prow, 0)
        rowb = (base0 + i * CH) // 16
        pltpu.async_copy(outb, s_out.at[pl.ds(rowb, CH // 16), :], sW).wait()

    slotA = (bufAA, bufBA)
    slotB = (bufAB, bufBB)
    semsA = (sAA, sBA)
    semsB = (sAB, sBB)
    dA = start(0, *slotA, *semsA)

    def pair(k, carry):
        j = 2 * k
        dB = start(j + 1, *slotB, *semsB)
        process(j, dA, *slotA, outbA)
        start(j + 2, *slotA, *semsA)
        process(j + 1, dB, *slotB, outbB)
        return carry

    lax.fori_loop(0, (NCHUNK - 1) // 2, pair, 0)
    process(NCHUNK - 1, dA, *slotA, outbA)


def kernel(node_features, edge_index, edge_features, msg_W, msg_b,
           W_ih, W_hh, b_ih, b_hh, c1_W, c1_b, c2_W, c2_b):
    src = edge_index[0]
    tgt = edge_index[1]
    nf_pad = jnp.pad(node_features, ((0, NP - N), (0, 0)))

    ef_p = jnp.reshape(edge_features, (E // 16, 16 * DE))

    wst = jnp.concatenate([msg_W[:, :D].T, msg_W[:, D:2 * D].T], axis=1)
    wet = msg_W[:, 2 * D:].T
    bd_we = jax.scipy.linalg.block_diag(*([wet] * 8))
    mb8 = jnp.tile(msg_b, 8).reshape(1, 8 * D)
    wih_t = W_ih.T
    whh_t = W_hh.T
    cst = jnp.concatenate([c1_W[:, :D].T, c1_W[:, D:2 * D].T], axis=1)
    cet = c1_W[:, 2 * D:].T
    c2wt = c2_W.T
    bd_c1 = jax.scipy.linalg.block_diag(*([cet] * 16))
    c1b8 = jnp.tile(c1_b, 16).reshape(1, 16 * DE)
    bd_c2 = jax.scipy.linalg.block_diag(*([c2wt] * 16))
    c2b8 = jnp.tile(c2_b, 16).reshape(1, 16 * NCLS)

    xs, xt = pl.pallas_call(
        _pre_nodes_body,
        grid=(NP // BN,),
        in_specs=[pl.BlockSpec((BN, D), lambda i: (i, 0)),
                  pl.BlockSpec((D, 2 * D), lambda i: (0, 0))],
        out_specs=[pl.BlockSpec((BN, D), lambda i: (i, 0)),
                   pl.BlockSpec((BN, D), lambda i: (i, 0))],
        out_shape=[jax.ShapeDtypeStruct((NP, D), jnp.float32),
                   jax.ShapeDtypeStruct((NP, D), jnp.float32)],
    )(nf_pad, wst)

    be8 = 1000
    eft = edge_features.T
    bep = 6400
    p = pl.pallas_call(
        _pre_edges_body,
        grid=(E // bep,),
        in_specs=[pl.BlockSpec((DE, bep), lambda i: (0, i)),
                  pl.BlockSpec((DE, D), lambda i: (0, 0)),
                  pl.BlockSpec((1, D), lambda i: (0, 0))],
        out_specs=pl.BlockSpec((bep, D), lambda i: (i, 0)),
        out_shape=jax.ShapeDtypeStruct((E, D), jnp.float32),
        compiler_params=pltpu.CompilerParams(
            fuse_transposed_lhs_in_matmul=True),
    )(eft, wet, msg_b.reshape(1, D))

    agg0, agg1 = _edge_a(xs, xt, p, src, tgt)

    ys, yt = pl.pallas_call(
        _gru_body,
        grid=(NP // BN,),
        in_specs=[pl.BlockSpec((BN, D), lambda i: (i, 0)),
                  pl.BlockSpec((BN, D), lambda i: (i, 0)),
                  pl.BlockSpec((BN, D), lambda i: (i, 0)),
                  pl.BlockSpec((D, 3 * D), lambda i: (0, 0)),
                  pl.BlockSpec((D, 3 * D), lambda i: (0, 0)),
                  pl.BlockSpec((1, 3 * D), lambda i: (0, 0)),
                  pl.BlockSpec((1, 3 * D), lambda i: (0, 0)),
                  pl.BlockSpec((D, 2 * DE), lambda i: (0, 0))],
        out_specs=[pl.BlockSpec((BN, DE), lambda i: (i, 0)),
                   pl.BlockSpec((BN, DE), lambda i: (i, 0))],
        out_shape=[jax.ShapeDtypeStruct((NP, DE), jnp.float32),
                   jax.ShapeDtypeStruct((NP, DE), jnp.float32)],
    )(agg0, agg1, nf_pad, wih_t, whh_t,
      b_ih.reshape(1, 3 * D), b_hh.reshape(1, 3 * D), cst)

    s = _edge_b(ys, yt, src, tgt)

    lp = pl.pallas_call(
        _cls_body,
        grid=(E // 16 // be8,),
        in_specs=[pl.BlockSpec((be8, 16 * DE), lambda i: (i, 0)),
                  pl.BlockSpec((be8, 16 * DE), lambda i: (i, 0)),
                  pl.BlockSpec((16 * DE, 16 * DE), lambda i: (0, 0)),
                  pl.BlockSpec((1, 16 * DE), lambda i: (0, 0)),
                  pl.BlockSpec((16 * DE, 16 * NCLS), lambda i: (0, 0)),
                  pl.BlockSpec((1, 16 * NCLS), lambda i: (0, 0))],
        out_specs=pl.BlockSpec((be8, 16 * NCLS), lambda i: (i, 0)),
        out_shape=jax.ShapeDtypeStruct((E // 16, 16 * NCLS), jnp.float32),
    )(s, ef_p, bd_c1, c1b8, bd_c2, c2b8)

    return jnp.reshape(lp, (E, NCLS))
